# baseline (speedup 1.0000x reference)
import functools

import jax
import jax.numpy as jnp
from jax import lax
from jax.experimental import pallas as pl
from jax.experimental.pallas import tpu as pltpu
from jax.experimental.pallas import tpu_sc as plsc

VOCAB = 1000000
D = 64
B = 16384

NC = 2
NS = 16
NW = NC * NS
L = 16

B_PER_W = B // NW
BCHUNK = 32
NBCHUNK = B_PER_W // BCHUNK
CHUNK = 128
NCHUNK = B_PER_W // CHUNK
NGROUP = B_PER_W // L


@functools.partial(
    pl.kernel,
    out_type=jax.ShapeDtypeStruct((B,), jnp.float32),
    mesh=plsc.VectorSubcoreMesh(core_axis_name="c", subcore_axis_name="s"),
    compiler_params=pltpu.CompilerParams(
        needs_layout_passes=False, use_tc_tiling_on_sc=True),
    scratch_types=[
        pltpu.VMEM((B_PER_W,), jnp.int32),
        pltpu.VMEM((B_PER_W,), jnp.int32),
        pltpu.VMEM((BCHUNK, 8, D), jnp.float32),
        pltpu.VMEM((BCHUNK, 8, D), jnp.float32),
        pltpu.VMEM((B_PER_W,), jnp.float32),
        pltpu.SemaphoreType.DMA,
    ],
)
def _dot_sc(tid_hbm, cid_hbm, w_hbm, c_hbm, out_hbm,
            tid_v, cid_v, w_blk, c_blk, out_v, sem):
    wid = lax.axis_index("s") * NC + lax.axis_index("c")
    base = pl.multiple_of(wid * B_PER_W, B_PER_W)
    pltpu.sync_copy(tid_hbm.at[wid], tid_v)
    pltpu.sync_copy(cid_hbm.at[wid], cid_v)
    iota16 = lax.iota(jnp.int32, L)

    def chunk_body(co, carry):
        o = pl.multiple_of(co * BCHUNK, BCHUNK)
        for g in range(BCHUNK // L):
            tb16 = (tid_v[pl.ds(o + g * L, L)] >> 3) * 8
            cb16 = (cid_v[pl.ds(o + g * L, L)] >> 3) * 8
            for k in range(L):
                i = g * L + k
                bt = pl.multiple_of(tb16[k], 8)
                bc = pl.multiple_of(cb16[k], 8)
                pltpu.async_copy(w_hbm.at[pl.ds(bt, 8), :],
                                 w_blk.at[i], sem)
                pltpu.async_copy(c_hbm.at[pl.ds(bc, 8), :],
                                 c_blk.at[i], sem)
        for i in range(BCHUNK):
            pltpu.make_async_copy(w_hbm.at[pl.ds(0, 8), :],
                                  w_blk.at[i], sem).wait()
            pltpu.make_async_copy(c_hbm.at[pl.ds(0, 8), :],
                                  c_blk.at[i], sem).wait()
        for g in range(BCHUNK // L):
            go = pl.multiple_of(co * BCHUNK + g * L, L)
            rows = g * L + iota16
            tr = tid_v[pl.ds(go, L)] & 7
            cr = cid_v[pl.ds(go, L)] & 7
            acc = jnp.zeros((L,), jnp.float32)
            for j in range(D):
                colj = jnp.full((L,), j, jnp.int32)
                acc = acc + (plsc.load_gather(w_blk, [rows, tr, colj])
                             * plsc.load_gather(c_blk, [rows, cr, colj]))
            out_v[pl.ds(go, L)] = acc
        return carry

    lax.fori_loop(0, NBCHUNK, chunk_body, 0)
    pltpu.sync_copy(out_v, out_hbm.at[pl.ds(base, B_PER_W)])


@functools.partial(
    pl.kernel,
    out_type=jax.ShapeDtypeStruct((B,), jnp.float32),
    mesh=plsc.VectorSubcoreMesh(core_axis_name="c", subcore_axis_name="s"),
    compiler_params=pltpu.CompilerParams(
        needs_layout_passes=False, use_tc_tiling_on_sc=False),
    scratch_types=[
        pltpu.VMEM((NCHUNK, CHUNK), jnp.int32),
        pltpu.VMEM((NCHUNK, CHUNK), jnp.int32),
        pltpu.VMEM((B_PER_W,), jnp.float32),
        pltpu.VMEM((B_PER_W,), jnp.float32),
        pltpu.VMEM((B_PER_W,), jnp.float32),
        pltpu.SemaphoreType.DMA,
    ],
)
def _bias_sc(tid_hbm, cid_hbm, wb_hbm, cb_hbm, dots_hbm, out_hbm,
             tid_v, cid_v, wb_v, cb_v, dots_v, sem):
    wid = lax.axis_index("s") * NC + lax.axis_index("c")
    base = pl.multiple_of(wid * B_PER_W, B_PER_W)
    crow0 = wid * NCHUNK
    pltpu.sync_copy(tid_hbm.at[pl.ds(crow0, NCHUNK)], tid_v)
    pltpu.sync_copy(cid_hbm.at[pl.ds(crow0, NCHUNK)], cid_v)
    pltpu.sync_copy(dots_hbm.at[pl.ds(base, B_PER_W)], dots_v)
    copies = []
    for k in range(NCHUNK):
        o = k * CHUNK
        copies.append(pltpu.async_copy(wb_hbm.at[tid_v.at[k]],
                                       wb_v.at[pl.ds(o, CHUNK)], sem))
        copies.append(pltpu.async_copy(cb_hbm.at[cid_v.at[k]],
                                       cb_v.at[pl.ds(o, CHUNK)], sem))
    for cp in copies:
        cp.wait()
    for g in range(NGROUP):
        o = g * L
        dots_v[pl.ds(o, L)] = (dots_v[pl.ds(o, L)]
                               + wb_v[pl.ds(o, L)] + cb_v[pl.ds(o, L)])
    pltpu.sync_copy(dots_v, out_hbm.at[pl.ds(base, B_PER_W)])


def kernel(target_ids, context_ids, w_emb, c_emb, w_bias, c_bias):
    tid = target_ids.astype(jnp.int32)
    cid = context_ids.astype(jnp.int32)
    zero = lax.optimization_barrier(jnp.zeros((1,), jnp.float32))
    dots = _dot_sc(tid.reshape(NW, B_PER_W), cid.reshape(NW, B_PER_W),
                   w_emb + zero, c_emb)
    return _bias_sc(tid.reshape(NW * NCHUNK, CHUNK),
                    cid.reshape(NW * NCHUNK, CHUNK),
                    w_bias.reshape(VOCAB), c_bias.reshape(VOCAB), dots)


# speedup vs baseline: 1.6420x; 1.6420x over previous
import functools

import jax
import jax.numpy as jnp
from jax import lax
from jax.experimental import pallas as pl
from jax.experimental.pallas import tpu as pltpu
from jax.experimental.pallas import tpu_sc as plsc

VOCAB = 1000000
D = 64
B = 16384

NC = 2
NS = 16
NW = NC * NS
L = 16

B_PER_W = B // NW
BCHUNK = 32
NBCHUNK = B_PER_W // BCHUNK
CHUNK = 128
NCHUNK = B_PER_W // CHUNK
NGROUP = B_PER_W // L


@functools.partial(
    pl.kernel,
    out_type=jax.ShapeDtypeStruct((B,), jnp.float32),
    mesh=plsc.VectorSubcoreMesh(core_axis_name="c", subcore_axis_name="s"),
    compiler_params=pltpu.CompilerParams(
        needs_layout_passes=False, use_tc_tiling_on_sc=True),
    scratch_types=[
        pltpu.VMEM((B_PER_W,), jnp.int32),
        pltpu.VMEM((B_PER_W,), jnp.int32),
        pltpu.VMEM((B_PER_W,), jnp.float32),
        pltpu.VMEM((BCHUNK, 8, D), jnp.float32),
        pltpu.VMEM((BCHUNK, 8, D), jnp.float32),
        pltpu.VMEM((B_PER_W,), jnp.float32),
        pltpu.SemaphoreType.DMA,
    ],
)
def _dot_sc(tid_hbm, cid_hbm, w_hbm, c_hbm, bsum_hbm, out_hbm,
            tid_v, cid_v, bsum_v, w_blk, c_blk, out_v, sem):
    wid = lax.axis_index("s") * NC + lax.axis_index("c")
    base = pl.multiple_of(wid * B_PER_W, B_PER_W)
    pltpu.sync_copy(tid_hbm.at[wid], tid_v)
    pltpu.sync_copy(cid_hbm.at[wid], cid_v)
    pltpu.sync_copy(bsum_hbm.at[pl.ds(base, B_PER_W)], bsum_v)
    iota16 = lax.iota(jnp.int32, L)

    def chunk_body(co, carry):
        o = pl.multiple_of(co * BCHUNK, BCHUNK)
        for g in range(BCHUNK // L):
            tb16 = (tid_v[pl.ds(o + g * L, L)] >> 3) * 8
            cb16 = (cid_v[pl.ds(o + g * L, L)] >> 3) * 8
            for k in range(L):
                i = g * L + k
                bt = pl.multiple_of(tb16[k], 8)
                bc = pl.multiple_of(cb16[k], 8)
                pltpu.async_copy(w_hbm.at[pl.ds(bt, 8), :],
                                 w_blk.at[i], sem)
                pltpu.async_copy(c_hbm.at[pl.ds(bc, 8), :],
                                 c_blk.at[i], sem)
        for i in range(BCHUNK):
            pltpu.make_async_copy(w_hbm.at[pl.ds(0, 8), :],
                                  w_blk.at[i], sem).wait()
            pltpu.make_async_copy(c_hbm.at[pl.ds(0, 8), :],
                                  c_blk.at[i], sem).wait()
        for g in range(BCHUNK // L):
            go = pl.multiple_of(co * BCHUNK + g * L, L)
            rows = g * L + iota16
            tr = tid_v[pl.ds(go, L)] & 7
            cr = cid_v[pl.ds(go, L)] & 7
            acc = bsum_v[pl.ds(go, L)]
            for j in range(D):
                colj = jnp.full((L,), j, jnp.int32)
                acc = acc + (plsc.load_gather(w_blk, [rows, tr, colj])
                             * plsc.load_gather(c_blk, [rows, cr, colj]))
            out_v[pl.ds(go, L)] = acc
        return carry

    lax.fori_loop(0, NBCHUNK, chunk_body, 0)
    pltpu.sync_copy(out_v, out_hbm.at[pl.ds(base, B_PER_W)])


@functools.partial(
    pl.kernel,
    out_type=jax.ShapeDtypeStruct((B,), jnp.float32),
    mesh=plsc.VectorSubcoreMesh(core_axis_name="c", subcore_axis_name="s"),
    compiler_params=pltpu.CompilerParams(
        needs_layout_passes=False, use_tc_tiling_on_sc=False),
    scratch_types=[
        pltpu.VMEM((NCHUNK, CHUNK), jnp.int32),
        pltpu.VMEM((NCHUNK, CHUNK), jnp.int32),
        pltpu.VMEM((B_PER_W,), jnp.float32),
        pltpu.VMEM((B_PER_W,), jnp.float32),
        pltpu.SemaphoreType.DMA,
    ],
)
def _bias_sc(tid_hbm, cid_hbm, wb_hbm, cb_hbm, out_hbm,
             tid_v, cid_v, wb_v, cb_v, sem):
    wid = lax.axis_index("s") * NC + lax.axis_index("c")
    base = pl.multiple_of(wid * B_PER_W, B_PER_W)
    crow0 = wid * NCHUNK
    pltpu.sync_copy(tid_hbm.at[pl.ds(crow0, NCHUNK)], tid_v)
    pltpu.sync_copy(cid_hbm.at[pl.ds(crow0, NCHUNK)], cid_v)
    copies = []
    for k in range(NCHUNK):
        o = k * CHUNK
        copies.append(pltpu.async_copy(wb_hbm.at[tid_v.at[k]],
                                       wb_v.at[pl.ds(o, CHUNK)], sem))
        copies.append(pltpu.async_copy(cb_hbm.at[cid_v.at[k]],
                                       cb_v.at[pl.ds(o, CHUNK)], sem))
    for cp in copies:
        cp.wait()
    for g in range(NGROUP):
        o = g * L
        wb_v[pl.ds(o, L)] = wb_v[pl.ds(o, L)] + cb_v[pl.ds(o, L)]
    pltpu.sync_copy(wb_v, out_hbm.at[pl.ds(base, B_PER_W)])


def kernel(target_ids, context_ids, w_emb, c_emb, w_bias, c_bias):
    tid = target_ids.astype(jnp.int32)
    cid = context_ids.astype(jnp.int32)
    bsum = _bias_sc(tid.reshape(NW * NCHUNK, CHUNK),
                    cid.reshape(NW * NCHUNK, CHUNK),
                    w_bias.reshape(VOCAB), c_bias.reshape(VOCAB))
    w_rm = lax.transpose(lax.optimization_barrier(w_emb.T), (1, 0))
    c_rm = lax.transpose(lax.optimization_barrier(c_emb.T), (1, 0))
    return _dot_sc(tid.reshape(NW, B_PER_W), cid.reshape(NW, B_PER_W),
                   w_rm, c_rm, bsum)


# speedup vs baseline: 1.7025x; 1.0368x over previous
import functools

import jax
import jax.numpy as jnp
from jax import lax
from jax.experimental import pallas as pl
from jax.experimental.pallas import tpu as pltpu
from jax.experimental.pallas import tpu_sc as plsc

VOCAB = 1000000
D = 64
B = 16384

NC = 2
NS = 16
NW = NC * NS
L = 16

B_PER_W = B // NW
BCHUNK = 16
NBCHUNK = B_PER_W // BCHUNK
CHUNK = 128
NCHUNK = B_PER_W // CHUNK
NGROUP = B_PER_W // L


@functools.partial(
    pl.kernel,
    out_type=jax.ShapeDtypeStruct((B,), jnp.float32),
    mesh=plsc.VectorSubcoreMesh(core_axis_name="c", subcore_axis_name="s"),
    compiler_params=pltpu.CompilerParams(
        needs_layout_passes=False, use_tc_tiling_on_sc=True),
    scratch_types=[
        pltpu.VMEM((B_PER_W,), jnp.int32),
        pltpu.VMEM((B_PER_W,), jnp.int32),
        pltpu.VMEM((B_PER_W,), jnp.float32),
        pltpu.VMEM((2, BCHUNK, 8, D), jnp.float32),
        pltpu.VMEM((2, BCHUNK, 8, D), jnp.float32),
        pltpu.VMEM((B_PER_W,), jnp.float32),
        pltpu.SemaphoreType.DMA,
        pltpu.SemaphoreType.DMA,
    ],
)
def _dot_sc(tid_hbm, cid_hbm, w_hbm, c_hbm, bsum_hbm, out_hbm,
            tid_v, cid_v, bsum_v, w_blk, c_blk, out_v, semA, semB):
    wid = lax.axis_index("s") * NC + lax.axis_index("c")
    base = pl.multiple_of(wid * B_PER_W, B_PER_W)
    pltpu.sync_copy(tid_hbm.at[wid], tid_v)
    pltpu.sync_copy(cid_hbm.at[wid], cid_v)
    pltpu.sync_copy(bsum_hbm.at[pl.ds(base, B_PER_W)], bsum_v)
    iota16 = lax.iota(jnp.int32, L)
    sems = (semA, semB)

    def fire(co, p, sem):
        o = pl.multiple_of(co * BCHUNK, BCHUNK)
        for g in range(BCHUNK // L):
            tb16 = (tid_v[pl.ds(o + g * L, L)] >> 3) * 8
            cb16 = (cid_v[pl.ds(o + g * L, L)] >> 3) * 8
            for k in range(L):
                i = g * L + k
                bt = pl.multiple_of(tb16[k], 8)
                bc = pl.multiple_of(cb16[k], 8)
                pltpu.async_copy(w_hbm.at[pl.ds(bt, 8), :],
                                 w_blk.at[p, i], sem)
                pltpu.async_copy(c_hbm.at[pl.ds(bc, 8), :],
                                 c_blk.at[p, i], sem)

    def drain(p, sem):
        for i in range(BCHUNK):
            pltpu.make_async_copy(w_hbm.at[pl.ds(0, 8), :],
                                  w_blk.at[p, i], sem).wait()
            pltpu.make_async_copy(c_hbm.at[pl.ds(0, 8), :],
                                  c_blk.at[p, i], sem).wait()

    def compute(co, p):
        for g in range(BCHUNK // L):
            go = pl.multiple_of(co * BCHUNK + g * L, L)
            rows = g * L + iota16
            tr = tid_v[pl.ds(go, L)] & 7
            cr = cid_v[pl.ds(go, L)] & 7
            acc0 = bsum_v[pl.ds(go, L)]
            acc1 = jnp.zeros((L,), jnp.float32)
            for j in range(0, D, 2):
                c0 = jnp.full((L,), j, jnp.int32)
                c1 = jnp.full((L,), j + 1, jnp.int32)
                acc0 = acc0 + (plsc.load_gather(w_blk.at[p], [rows, tr, c0])
                               * plsc.load_gather(c_blk.at[p], [rows, cr, c0]))
                acc1 = acc1 + (plsc.load_gather(w_blk.at[p], [rows, tr, c1])
                               * plsc.load_gather(c_blk.at[p], [rows, cr, c1]))
            out_v[pl.ds(go, L)] = acc0 + acc1

    fire(0, 0, semA)

    def pair_body(it, carry):
        co = it * 2
        for p in range(2):
            drain(p, sems[p])
            nxt = co + p + 1

            @pl.when(nxt < NBCHUNK)
            def _():
                fire(nxt, 1 - p, sems[1 - p])

            compute(co + p, p)
        return carry

    lax.fori_loop(0, NBCHUNK // 2, pair_body, 0)
    pltpu.sync_copy(out_v, out_hbm.at[pl.ds(base, B_PER_W)])


@functools.partial(
    pl.kernel,
    out_type=jax.ShapeDtypeStruct((B,), jnp.float32),
    mesh=plsc.VectorSubcoreMesh(core_axis_name="c", subcore_axis_name="s"),
    compiler_params=pltpu.CompilerParams(
        needs_layout_passes=False, use_tc_tiling_on_sc=False),
    scratch_types=[
        pltpu.VMEM((NCHUNK, CHUNK), jnp.int32),
        pltpu.VMEM((NCHUNK, CHUNK), jnp.int32),
        pltpu.VMEM((B_PER_W,), jnp.float32),
        pltpu.VMEM((B_PER_W,), jnp.float32),
        pltpu.SemaphoreType.DMA,
    ],
)
def _bias_sc(tid_hbm, cid_hbm, wb_hbm, cb_hbm, out_hbm,
             tid_v, cid_v, wb_v, cb_v, sem):
    wid = lax.axis_index("s") * NC + lax.axis_index("c")
    base = pl.multiple_of(wid * B_PER_W, B_PER_W)
    crow0 = wid * NCHUNK
    pltpu.sync_copy(tid_hbm.at[pl.ds(crow0, NCHUNK)], tid_v)
    pltpu.sync_copy(cid_hbm.at[pl.ds(crow0, NCHUNK)], cid_v)
    copies = []
    for k in range(NCHUNK):
        o = k * CHUNK
        copies.append(pltpu.async_copy(wb_hbm.at[tid_v.at[k]],
                                       wb_v.at[pl.ds(o, CHUNK)], sem))
        copies.append(pltpu.async_copy(cb_hbm.at[cid_v.at[k]],
                                       cb_v.at[pl.ds(o, CHUNK)], sem))
    for cp in copies:
        cp.wait()
    for g in range(NGROUP):
        o = g * L
        wb_v[pl.ds(o, L)] = wb_v[pl.ds(o, L)] + cb_v[pl.ds(o, L)]
    pltpu.sync_copy(wb_v, out_hbm.at[pl.ds(base, B_PER_W)])


def kernel(target_ids, context_ids, w_emb, c_emb, w_bias, c_bias):
    tid = target_ids.astype(jnp.int32)
    cid = context_ids.astype(jnp.int32)
    bsum = _bias_sc(tid.reshape(NW * NCHUNK, CHUNK),
                    cid.reshape(NW * NCHUNK, CHUNK),
                    w_bias.reshape(VOCAB), c_bias.reshape(VOCAB))
    w_rm = lax.transpose(lax.optimization_barrier(w_emb.T), (1, 0))
    c_rm = lax.transpose(lax.optimization_barrier(c_emb.T), (1, 0))
    return _dot_sc(tid.reshape(NW, B_PER_W), cid.reshape(NW, B_PER_W),
                   w_rm, c_rm, bsum)


# speedup vs baseline: 1.7035x; 1.0006x over previous
import functools

import jax
import jax.numpy as jnp
from jax import lax
from jax.experimental import pallas as pl
from jax.experimental.pallas import tpu as pltpu
from jax.experimental.pallas import tpu_sc as plsc

VOCAB = 1000000
D = 64
B = 16384

NC = 2
NS = 16
NW = NC * NS
L = 16

B_PER_W = B // NW
BCHUNK = 16
NBCHUNK = B_PER_W // BCHUNK
CHUNK = 128
NCHUNK = B_PER_W // CHUNK
NGROUP = B_PER_W // L


@functools.partial(
    pl.kernel,
    out_type=jax.ShapeDtypeStruct((B,), jnp.float32),
    mesh=plsc.VectorSubcoreMesh(core_axis_name="c", subcore_axis_name="s"),
    compiler_params=pltpu.CompilerParams(
        needs_layout_passes=False, use_tc_tiling_on_sc=True),
    scratch_types=[
        pltpu.VMEM((B_PER_W,), jnp.int32),
        pltpu.VMEM((B_PER_W,), jnp.int32),
        pltpu.VMEM((B_PER_W,), jnp.float32),
        pltpu.VMEM((2, BCHUNK * 8, D), jnp.float32),
        pltpu.VMEM((2, BCHUNK * 8, D), jnp.float32),
        pltpu.VMEM((B_PER_W,), jnp.float32),
        pltpu.SemaphoreType.DMA,
        pltpu.SemaphoreType.DMA,
    ],
)
def _dot_sc(tid_hbm, cid_hbm, w_hbm, c_hbm, bsum_hbm, out_hbm,
            tid_v, cid_v, bsum_v, w_blk, c_blk, out_v, semA, semB):
    wid = lax.axis_index("s") * NC + lax.axis_index("c")
    base = pl.multiple_of(wid * B_PER_W, B_PER_W)
    pltpu.sync_copy(tid_hbm.at[wid], tid_v)
    pltpu.sync_copy(cid_hbm.at[wid], cid_v)
    pltpu.sync_copy(bsum_hbm.at[pl.ds(base, B_PER_W)], bsum_v)
    iota16 = lax.iota(jnp.int32, L)
    sems = (semA, semB)

    def fire(co, p, sem):
        o = pl.multiple_of(co * BCHUNK, BCHUNK)
        for g in range(BCHUNK // L):
            tb16 = (tid_v[pl.ds(o + g * L, L)] >> 3) * 8
            cb16 = (cid_v[pl.ds(o + g * L, L)] >> 3) * 8
            for k in range(L):
                i = g * L + k
                bt = pl.multiple_of(tb16[k], 8)
                bc = pl.multiple_of(cb16[k], 8)
                pltpu.async_copy(w_hbm.at[pl.ds(bt, 8), :],
                                 w_blk.at[p, pl.ds(i * 8, 8), :], sem)
                pltpu.async_copy(c_hbm.at[pl.ds(bc, 8), :],
                                 c_blk.at[p, pl.ds(i * 8, 8), :], sem)

    def drain(p, sem):
        for i in range(BCHUNK):
            pltpu.make_async_copy(w_hbm.at[pl.ds(0, 8), :],
                                  w_blk.at[p, pl.ds(i * 8, 8), :], sem).wait()
            pltpu.make_async_copy(c_hbm.at[pl.ds(0, 8), :],
                                  c_blk.at[p, pl.ds(i * 8, 8), :], sem).wait()

    def compute(co, p):
        for g in range(BCHUNK // L):
            go = pl.multiple_of(co * BCHUNK + g * L, L)
            rows = (g * L + iota16) * 8
            rt = rows + (tid_v[pl.ds(go, L)] & 7)
            rc = rows + (cid_v[pl.ds(go, L)] & 7)
            acc0 = bsum_v[pl.ds(go, L)]
            acc1 = jnp.zeros((L,), jnp.float32)
            for j in range(0, D, 2):
                c0 = jnp.full((L,), j, jnp.int32)
                c1 = jnp.full((L,), j + 1, jnp.int32)
                acc0 = acc0 + (plsc.load_gather(w_blk.at[p], [rt, c0])
                               * plsc.load_gather(c_blk.at[p], [rc, c0]))
                acc1 = acc1 + (plsc.load_gather(w_blk.at[p], [rt, c1])
                               * plsc.load_gather(c_blk.at[p], [rc, c1]))
            out_v[pl.ds(go, L)] = acc0 + acc1

    fire(0, 0, semA)

    def pair_body(it, carry):
        co = it * 2
        for p in range(2):
            drain(p, sems[p])
            nxt = co + p + 1

            @pl.when(nxt < NBCHUNK)
            def _():
                fire(nxt, 1 - p, sems[1 - p])

            compute(co + p, p)
        return carry

    lax.fori_loop(0, NBCHUNK // 2, pair_body, 0)
    pltpu.sync_copy(out_v, out_hbm.at[pl.ds(base, B_PER_W)])


@functools.partial(
    pl.kernel,
    out_type=jax.ShapeDtypeStruct((B,), jnp.float32),
    mesh=plsc.VectorSubcoreMesh(core_axis_name="c", subcore_axis_name="s"),
    compiler_params=pltpu.CompilerParams(
        needs_layout_passes=False, use_tc_tiling_on_sc=False),
    scratch_types=[
        pltpu.VMEM((NCHUNK, CHUNK), jnp.int32),
        pltpu.VMEM((NCHUNK, CHUNK), jnp.int32),
        pltpu.VMEM((B_PER_W,), jnp.float32),
        pltpu.VMEM((B_PER_W,), jnp.float32),
        pltpu.SemaphoreType.DMA,
    ],
)
def _bias_sc(tid_hbm, cid_hbm, wb_hbm, cb_hbm, out_hbm,
             tid_v, cid_v, wb_v, cb_v, sem):
    wid = lax.axis_index("s") * NC + lax.axis_index("c")
    base = pl.multiple_of(wid * B_PER_W, B_PER_W)
    crow0 = wid * NCHUNK
    pltpu.sync_copy(tid_hbm.at[pl.ds(crow0, NCHUNK)], tid_v)
    pltpu.sync_copy(cid_hbm.at[pl.ds(crow0, NCHUNK)], cid_v)
    copies = []
    for k in range(NCHUNK):
        o = k * CHUNK
        copies.append(pltpu.async_copy(wb_hbm.at[tid_v.at[k]],
                                       wb_v.at[pl.ds(o, CHUNK)], sem))
        copies.append(pltpu.async_copy(cb_hbm.at[cid_v.at[k]],
                                       cb_v.at[pl.ds(o, CHUNK)], sem))
    for cp in copies:
        cp.wait()
    for g in range(NGROUP):
        o = g * L
        wb_v[pl.ds(o, L)] = wb_v[pl.ds(o, L)] + cb_v[pl.ds(o, L)]
    pltpu.sync_copy(wb_v, out_hbm.at[pl.ds(base, B_PER_W)])


def kernel(target_ids, context_ids, w_emb, c_emb, w_bias, c_bias):
    tid = target_ids.astype(jnp.int32)
    cid = context_ids.astype(jnp.int32)
    bsum = _bias_sc(tid.reshape(NW * NCHUNK, CHUNK),
                    cid.reshape(NW * NCHUNK, CHUNK),
                    w_bias.reshape(VOCAB), c_bias.reshape(VOCAB))
    w_rm = lax.transpose(lax.optimization_barrier(w_emb.T), (1, 0))
    c_rm = lax.transpose(lax.optimization_barrier(c_emb.T), (1, 0))
    return _dot_sc(tid.reshape(NW, B_PER_W), cid.reshape(NW, B_PER_W),
                   w_rm, c_rm, bsum)
